# baseline (speedup 1.0000x reference)
import functools

import jax
import jax.numpy as jnp
from jax import lax
from jax.experimental import pallas as pl
from jax.experimental.pallas import tpu as pltpu
from jax.experimental.pallas import tpu_sc as plsc

N_DRUGS = 1000
N_LINES = 100000
BATCH = 16384
N_CONC = 20

NC = 2
NS = 16
NW = NC * NS
BPW = BATCH // NW
CHUNK = 128
NCH = BPW // CHUNK
LANES = 16

_mesh = plsc.VectorSubcoreMesh(
    core_axis_name="c", subcore_axis_name="s", num_cores=NC, num_subcores=NS
)


@functools.partial(
    pl.kernel,
    out_type=jax.ShapeDtypeStruct((N_CONC, BATCH), jnp.float32),
    mesh=_mesh,
    scratch_types=[
        pltpu.VMEM((NCH, CHUNK), jnp.int32),
        pltpu.VMEM((NCH, CHUNK), jnp.int32),
        pltpu.VMEM((NCH, CHUNK), jnp.int32),
        pltpu.VMEM((BPW,), jnp.float32),
        pltpu.VMEM((BPW,), jnp.float32),
        pltpu.VMEM((BPW,), jnp.float32),
        pltpu.VMEM((BPW,), jnp.float32),
        pltpu.VMEM((BPW,), jnp.float32),
        pltpu.VMEM((N_CONC, BPW), jnp.float32),
        pltpu.VMEM((N_CONC, BPW), jnp.float32),
        pltpu.SemaphoreType.DMA,
    ],
)
def _sc_sigmoid(d_hbm, l_hbm, ct_hbm, b1_hbm, b2_hbm, b3_hbm, b4_hbm,
                b4ld_hbm, out_hbm,
                lidx, didx, fidx, b1v, b2v, b3v, b4v, b4ldv, cv, ov, sem):
    wid = lax.axis_index("s") * NC + lax.axis_index("c")
    base = wid * BPW

    pltpu.sync_copy(l_hbm.at[pl.ds(wid * NCH, NCH)], lidx)
    pltpu.sync_copy(d_hbm.at[pl.ds(wid * NCH, NCH)], didx)

    for k in range(NCH):
        for j in range(CHUNK // LANES):
            sl = pl.ds(j * LANES, LANES)
            fidx[k, sl] = lidx[k, sl] * N_DRUGS + didx[k, sl]

    copies = [pltpu.async_copy(
        ct_hbm.at[:, pl.ds(base, BPW)], cv, sem)]
    for k in range(NCH):
        dsl = pl.ds(k * CHUNK, CHUNK)
        copies.append(pltpu.async_copy(b1_hbm.at[lidx.at[k]], b1v.at[dsl], sem))
        copies.append(pltpu.async_copy(b2_hbm.at[lidx.at[k]], b2v.at[dsl], sem))
        copies.append(pltpu.async_copy(b3_hbm.at[lidx.at[k]], b3v.at[dsl], sem))
        copies.append(pltpu.async_copy(b4_hbm.at[lidx.at[k]], b4v.at[dsl], sem))
        copies.append(pltpu.async_copy(b4ld_hbm.at[fidx.at[k]], b4ldv.at[dsl], sem))
    for cp in copies:
        cp.wait()

    def group(g, carry):
        rsl = pl.ds(g * LANES, LANES)
        b1 = b1v[rsl]
        db = b2v[rsl] - b1
        b3 = b3v[rsl]
        b4 = b4v[rsl] + b4ldv[rsl]
        for j in range(N_CONC):
            t = b4 * (cv[j, rsl] - b3)
            ov[j, rsl] = b1 + db / (1.0 + jnp.exp(-t))
        return carry

    lax.fori_loop(0, BPW // LANES, group, 0)

    pltpu.sync_copy(ov, out_hbm.at[:, pl.ds(base, BPW)])


def kernel(d, l, c, B1l, B2l, B3l, B4l, B4ld):
    d2 = d.reshape(BATCH // CHUNK, CHUNK)
    l2 = l.reshape(BATCH // CHUNK, CHUNK)
    ct = c.T
    b4ld_flat = B4ld.reshape(-1)
    out_t = _sc_sigmoid(d2, l2, ct, B1l, B2l, B3l, B4l, b4ld_flat)
    return out_t.T


# speedup vs baseline: 5.2017x; 5.2017x over previous
import functools

import jax
import jax.numpy as jnp
from jax import lax
from jax.experimental import pallas as pl
from jax.experimental.pallas import tpu as pltpu
from jax.experimental.pallas import tpu_sc as plsc

N_DRUGS = 1000
N_LINES = 100000
BATCH = 16384
N_CONC = 20

NC = 2
NS = 16
NW = NC * NS
BPW = BATCH // NW
CHUNK = 128
NCH = BPW // CHUNK
LANES = 16
WMAX = (N_DRUGS // LANES - 1) * LANES
DMA_B = 16
N_WAVES = BPW // DMA_B

_mesh = plsc.VectorSubcoreMesh(
    core_axis_name="c", subcore_axis_name="s", num_cores=NC, num_subcores=NS
)


@functools.partial(
    pl.kernel,
    out_type=jax.ShapeDtypeStruct((N_CONC, BATCH), jnp.float32),
    mesh=_mesh,
    scratch_types=[
        pltpu.VMEM((NCH, CHUNK), jnp.int32),
        pltpu.VMEM((NCH, CHUNK), jnp.int32),
        pltpu.VMEM((BPW,), jnp.float32),
        pltpu.VMEM((BPW,), jnp.float32),
        pltpu.VMEM((BPW,), jnp.float32),
        pltpu.VMEM((BPW,), jnp.float32),
        pltpu.VMEM((NCH, CHUNK), jnp.int32),
        pltpu.VMEM_SHARED((NS * BPW * CHUNK,), jnp.float32),
        pltpu.VMEM((BPW,), jnp.float32),
        pltpu.VMEM((N_CONC, BPW), jnp.float32),
        pltpu.VMEM((N_CONC, BPW), jnp.float32),
        pltpu.SemaphoreType.DMA,
        pltpu.SemaphoreType.DMA,
    ],
)
def _sc_sigmoid(d_hbm, l_hbm, ct_hbm, b1_hbm, b2_hbm, b3_hbm, b4_hbm,
                b4ld_hbm, out_hbm,
                lidx, didx, b1v, b2v, b3v, b4v, fidx, lines, b4e,
                cv, ov, sem, wsem):
    wid = lax.axis_index("s") * NC + lax.axis_index("c")
    base = wid * BPW
    lane_iota = lax.iota(jnp.int32, LANES)

    pltpu.sync_copy(l_hbm.at[pl.ds(wid * NCH, NCH)], lidx)
    pltpu.sync_copy(d_hbm.at[pl.ds(wid * NCH, NCH)], didx)

    sid = lax.axis_index("s")
    sbase = sid * (BPW * CHUNK)
    for k in range(NCH):
        for j in range(CHUNK // LANES):
            sl = pl.ds(j * LANES, LANES)
            dv = didx[k, sl]
            i0 = k * CHUNK + j * LANES
            fidx[k, sl] = sbase + (lane_iota + i0) * CHUNK + (dv & (CHUNK - 1))

    copies = [pltpu.async_copy(ct_hbm.at[:, pl.ds(base, BPW)], cv, sem)]
    for k in range(NCH):
        dsl = pl.ds(k * CHUNK, CHUNK)
        copies.append(pltpu.async_copy(b1_hbm.at[lidx.at[k]], b1v.at[dsl], sem))
        copies.append(pltpu.async_copy(b2_hbm.at[lidx.at[k]], b2v.at[dsl], sem))
        copies.append(pltpu.async_copy(b3_hbm.at[lidx.at[k]], b3v.at[dsl], sem))
        copies.append(pltpu.async_copy(b4_hbm.at[lidx.at[k]], b4v.at[dsl], sem))

    def wave(q, carry):
        k = q // (CHUNK // DMA_B)
        j0 = (q - k * (CHUNK // DMA_B)) * DMA_B
        handles = []
        for s in range(DMA_B // LANES):
            lvec = lidx[k, pl.ds(j0 + s * LANES, LANES)]
            dvec = didx[k, pl.ds(j0 + s * LANES, LANES)]
            for u in range(LANES):
                i = q * DMA_B + s * LANES + u
                dt = (dvec[u] // CHUNK) * CHUNK
                handles.append(pltpu.async_copy(
                    b4ld_hbm.at[pl.ds(lvec[u], 1), pl.ds(dt, CHUNK)].at[0],
                    lines.at[pl.ds(sbase + i * CHUNK, CHUNK)], wsem))
        for h in handles:
            h.wait()
        return carry

    lax.fori_loop(0, N_WAVES, wave, 0)

    ecopies = []
    for k in range(NCH):
        dsl = pl.ds(k * CHUNK, CHUNK)
        ecopies.append(pltpu.async_copy(
            lines.at[fidx.at[k]], b4e.at[dsl], wsem))
    for cp in ecopies:
        cp.wait()

    for cp in copies:
        cp.wait()

    def group(g, carry):
        rsl = pl.ds(g * LANES, LANES)
        b1 = b1v[rsl]
        db = b2v[rsl] - b1
        b3 = b3v[rsl]
        b4 = b4v[rsl] + b4e[rsl]
        for j in range(N_CONC):
            t = b4 * (cv[j, rsl] - b3)
            ov[j, rsl] = b1 + db / (1.0 + jnp.exp(-t))
        return carry

    lax.fori_loop(0, BPW // LANES, group, 0)

    pltpu.sync_copy(ov, out_hbm.at[:, pl.ds(base, BPW)])


def kernel(d, l, c, B1l, B2l, B3l, B4l, B4ld):
    d2 = d.reshape(BATCH // CHUNK, CHUNK)
    l2 = l.reshape(BATCH // CHUNK, CHUNK)
    ct = c.T
    out_t = _sc_sigmoid(d2, l2, ct, B1l, B2l, B3l, B4l, B4ld)
    return out_t.T
